# Initial kernel scaffold; baseline (speedup 1.0000x reference)
import jax
import jax.numpy as jnp
from jax.experimental import pallas as pl


def kernel(edges, logit_X0, theta, mu, rho):
    raise NotImplementedError("write your pallas kernel here")


# speedup vs baseline: 17.4092x; 17.4092x over previous
import functools

import jax
import jax.numpy as jnp
from jax import lax
from jax.experimental import pallas as pl
from jax.experimental.pallas import tpu as pltpu
from jax.experimental.pallas import tpu_sc as plsc

_L = 16


def _diffusion_call(Tm1, E, N):
    mesh = plsc.VectorSubcoreMesh(
        core_axis_name="c", subcore_axis_name="s", num_cores=2, num_subcores=16)

    @functools.partial(
        pl.kernel,
        out_type=(
            jax.ShapeDtypeStruct((Tm1 + 1, N), jnp.float32),
            jax.ShapeDtypeStruct((Tm1, E), jnp.float32),
        ),
        mesh=mesh,
        compiler_params=pltpu.CompilerParams(needs_layout_passes=False),
        scratch_types=[
            pltpu.VMEM((N,), jnp.float32),
            pltpu.VMEM((N,), jnp.float32),
            pltpu.VMEM((E,), jnp.int32),
            pltpu.VMEM((E,), jnp.int32),
            pltpu.VMEM((E,), jnp.float32),
            pltpu.VMEM((E,), jnp.float32),
            pltpu.VMEM((_L,), jnp.float32),
            pltpu.VMEM((_L,), jnp.float32),
            pltpu.VMEM((_L,), jnp.float32),
        ],
    )
    def body(u_hbm, v_hbm, lx_hbm, th_hbm, mu_hbm, rho_hbm,
             x_out, kap_out,
             x_v, lx_v, ui_v, vi_v, d_v, kap_v, th_v, mu_v, rho_v):
        is_w0 = jnp.logical_and(
            lax.axis_index("c") == 0, lax.axis_index("s") == 0)

        @pl.when(is_w0)
        def _():
            pltpu.sync_copy(lx_hbm, lx_v)
            pltpu.sync_copy(th_hbm, th_v)
            pltpu.sync_copy(mu_hbm, mu_v)
            pltpu.sync_copy(rho_hbm, rho_v)
            eps = 1.0 / (1.0 + jnp.exp(-th_v[...]))
            mu = mu_v[...]
            rho = rho_v[...]

            def init_body(i, c):
                z = lx_v[pl.ds(i * _L, _L)]
                x_v[pl.ds(i * _L, _L)] = 1.0 / (1.0 + jnp.exp(-z))
                return c

            lax.fori_loop(0, N // _L, init_body, 0)
            pltpu.sync_copy(x_v, x_out.at[0])

            def step(t, c):
                pltpu.sync_copy(u_hbm.at[t], ui_v)
                pltpu.sync_copy(v_hbm.at[t], vi_v)

                def gat(j, c2):
                    iu = ui_v[pl.ds(j * _L, _L)]
                    iv = vi_v[pl.ds(j * _L, _L)]
                    dd = plsc.load_gather(x_v, [iu]) - plsc.load_gather(x_v, [iv])
                    d_v[pl.ds(j * _L, _L)] = dd
                    z = rho * (eps - jnp.abs(dd))
                    kap_v[pl.ds(j * _L, _L)] = 1.0 / (1.0 + jnp.exp(-z))
                    return c2

                lax.fori_loop(0, E // _L, gat, 0)

                def sca(j, c2):
                    iu = ui_v[pl.ds(j * _L, _L)]
                    iv = vi_v[pl.ds(j * _L, _L)]
                    dd = d_v[pl.ds(j * _L, _L)] * mu
                    plsc.addupdate_scatter(x_v, [iu], -dd)
                    plsc.addupdate_scatter(x_v, [iv], dd)
                    return c2

                lax.fori_loop(0, E // _L, sca, 0)
                pltpu.sync_copy(x_v, x_out.at[t + 1])
                pltpu.sync_copy(kap_v, kap_out.at[t])
                return c

            lax.fori_loop(0, Tm1, step, 0)

    return body


def kernel(edges, logit_X0, theta, mu, rho):
    Tm1, E, _ = edges.shape
    N = logit_X0.shape[0]
    u = edges[:, :, 0]
    v = edges[:, :, 1]
    th16 = jnp.broadcast_to(theta.astype(jnp.float32), (_L,))
    mu16 = jnp.full((_L,), mu, jnp.float32)
    rho16 = jnp.full((_L,), rho, jnp.float32)
    X, kap = _diffusion_call(Tm1, E, N)(
        u, v, logit_X0.astype(jnp.float32), th16, mu16, rho16)
    return X, kap.reshape(-1)


# speedup vs baseline: 34.9338x; 2.0066x over previous
import functools

import jax
import jax.numpy as jnp
from jax import lax
from jax.experimental import pallas as pl
from jax.experimental.pallas import tpu as pltpu
from jax.experimental.pallas import tpu_sc as plsc

_L = 16


def _diffusion_call(Tm1, E, N):
    mesh = plsc.VectorSubcoreMesh(
        core_axis_name="c", subcore_axis_name="s", num_cores=2, num_subcores=16)
    NS = 16
    KE = Tm1 * E
    KCH = KE // NS

    @functools.partial(
        pl.kernel,
        out_type=(
            jax.ShapeDtypeStruct((Tm1 + 1, N), jnp.float32),
            jax.ShapeDtypeStruct((KE,), jnp.float32),
            jax.ShapeDtypeStruct((KE,), jnp.float32),
        ),
        mesh=mesh,
        compiler_params=pltpu.CompilerParams(needs_layout_passes=False),
        scratch_types=[
            pltpu.VMEM((N,), jnp.float32),
            pltpu.VMEM((2, 2 * E), jnp.int32),
            pltpu.VMEM((E,), jnp.float32),
            pltpu.VMEM((_L,), jnp.float32),
            pltpu.VMEM((_L,), jnp.float32),
            pltpu.VMEM((_L,), jnp.float32),
            pltpu.SemaphoreType.DMA,
            pltpu.SemaphoreType.DMA,
            pltpu.SemaphoreType.DMA,
        ],
    )
    def body(uv_hbm, lx_hbm, th_hbm, mu_hbm, rho_hbm,
             x_out, kap_out, d_out,
             x_v, uv_v, dmu_v, th_v, mu_v, rho_v, sem_x, sem_uv, sem_d):
        cid = lax.axis_index("c")
        sid = lax.axis_index("s")

        @pl.when(jnp.logical_and(cid == 0, sid == 0))
        def _():
            pltpu.sync_copy(mu_hbm, mu_v)
            mu = mu_v[...]
            pltpu.sync_copy(lx_hbm, x_v)

            def init_body(i, c):
                z = x_v[pl.ds(i * _L, _L)]
                x_v[pl.ds(i * _L, _L)] = 1.0 / (1.0 + jnp.exp(-z))
                return c

            lax.fori_loop(0, N // _L, init_body, 0)
            pltpu.async_copy(x_v, x_out.at[0], sem_x)
            pltpu.sync_copy(uv_hbm.at[0], uv_v.at[0])

            def step(t, c):
                par = lax.rem(t, 2)
                npar = 1 - par

                @pl.when(t + 1 < Tm1)
                def _():
                    pltpu.async_copy(uv_hbm.at[t + 1], uv_v.at[npar], sem_uv)

                @pl.when(t > 0)
                def _():
                    pltpu.make_async_copy(
                        dmu_v, d_out.at[pl.ds(0, E)], sem_d).wait()

                def gat(j, c2):
                    iu = uv_v[par, pl.ds(j * _L, _L)]
                    iv = uv_v[par, pl.ds(E + j * _L, _L)]
                    dd = plsc.load_gather(x_v, [iu]) - plsc.load_gather(x_v, [iv])
                    dmu_v[pl.ds(j * _L, _L)] = dd * mu
                    return c2

                lax.fori_loop(0, E // _L, gat, 0)
                pltpu.async_copy(dmu_v, d_out.at[pl.ds(t * E, E)], sem_d)

                pltpu.make_async_copy(x_v, x_out.at[0], sem_x).wait()

                def sca(j, c2):
                    iu = uv_v[par, pl.ds(j * _L, _L)]
                    iv = uv_v[par, pl.ds(E + j * _L, _L)]
                    dm = dmu_v[pl.ds(j * _L, _L)]
                    plsc.addupdate_scatter(x_v, [iv], dm)
                    plsc.addupdate_scatter(x_v, [iu], -dm)
                    return c2

                lax.fori_loop(0, E // _L, sca, 0)
                pltpu.async_copy(x_v, x_out.at[t + 1], sem_x)

                @pl.when(t + 1 < Tm1)
                def _():
                    pltpu.make_async_copy(
                        uv_hbm.at[0], uv_v.at[npar], sem_uv).wait()

                return c

            lax.fori_loop(0, Tm1, step, 0)
            pltpu.make_async_copy(dmu_v, d_out.at[pl.ds(0, E)], sem_d).wait()
            pltpu.make_async_copy(x_v, x_out.at[0], sem_x).wait()

        @pl.when(cid == 0)
        def _():
            pltpu.sync_copy(th_hbm, th_v)
            pltpu.sync_copy(mu_hbm, mu_v)
            pltpu.sync_copy(rho_hbm, rho_v)
            eps = 1.0 / (1.0 + jnp.exp(-th_v[...]))
            bias = rho_v[...] * eps
            scale = rho_v[...] / mu_v[...]
            plsc.subcore_barrier()
            base = sid * KCH
            pltpu.sync_copy(d_out.at[pl.ds(base, KCH)], x_v.at[pl.ds(0, KCH)])

            def kap(i, c):
                dm = x_v[pl.ds(i * _L, _L)]
                z = bias - scale * jnp.abs(dm)
                x_v[pl.ds(i * _L, _L)] = 1.0 / (1.0 + jnp.exp(-z))
                return c

            lax.fori_loop(0, KCH // _L, kap, 0)
            pltpu.sync_copy(x_v.at[pl.ds(0, KCH)], kap_out.at[pl.ds(base, KCH)])

    return body


def kernel(edges, logit_X0, theta, mu, rho):
    Tm1, E, _ = edges.shape
    N = logit_X0.shape[0]
    uv = jnp.concatenate([edges[:, :, 0], edges[:, :, 1]], axis=1)
    th16 = jnp.broadcast_to(theta.astype(jnp.float32), (_L,))
    mu16 = jnp.full((_L,), mu, jnp.float32)
    rho16 = jnp.full((_L,), rho, jnp.float32)
    X, kap, _ = _diffusion_call(Tm1, E, N)(
        uv, logit_X0.astype(jnp.float32), th16, mu16, rho16)
    return X, kap


# speedup vs baseline: 78.4288x; 2.2451x over previous
import functools

import jax
import jax.numpy as jnp
from jax import lax
from jax.experimental import pallas as pl
from jax.experimental.pallas import tpu as pltpu
from jax.experimental.pallas import tpu_sc as plsc

_L = 16


def _diffusion_call(Tm1, E, N):
    mesh = plsc.VectorSubcoreMesh(
        core_axis_name="c", subcore_axis_name="s", num_cores=2, num_subcores=16)
    NS = 16
    KE = Tm1 * E
    KCH = KE // NS

    @functools.partial(
        pl.kernel,
        out_type=(
            jax.ShapeDtypeStruct((Tm1 + 1, N), jnp.float32),
            jax.ShapeDtypeStruct((KE,), jnp.float32),
            jax.ShapeDtypeStruct((KE,), jnp.float32),
        ),
        mesh=mesh,
        compiler_params=pltpu.CompilerParams(needs_layout_passes=False),
        scratch_types=[
            pltpu.VMEM((N,), jnp.float32),
            pltpu.VMEM((2, 2 * E), jnp.int32),
            pltpu.VMEM((E,), jnp.float32),
            pltpu.VMEM((_L,), jnp.float32),
            pltpu.VMEM((_L,), jnp.float32),
            pltpu.VMEM((_L,), jnp.float32),
            pltpu.SemaphoreType.DMA,
            pltpu.SemaphoreType.DMA,
            pltpu.SemaphoreType.DMA,
        ],
    )
    def body(uv_hbm, lx_hbm, th_hbm, mu_hbm, rho_hbm,
             x_out, kap_out, d_out,
             x_v, uv_v, dmu_v, th_v, mu_v, rho_v, sem_x, sem_uv, sem_d):
        cid = lax.axis_index("c")
        sid = lax.axis_index("s")

        @pl.when(jnp.logical_and(cid == 0, sid == 0))
        def _():
            pltpu.sync_copy(mu_hbm, mu_v)
            mu = mu_v[...]
            pltpu.sync_copy(lx_hbm, x_v)

            @plsc.parallel_loop(0, N, _L, unroll=4)
            def _(i):
                z = x_v[pl.ds(i, _L)]
                x_v[pl.ds(i, _L)] = 1.0 / (1.0 + jnp.exp(-z))
            pltpu.async_copy(x_v, x_out.at[0], sem_x)
            pltpu.sync_copy(uv_hbm.at[0], uv_v.at[0])

            def step(t, c):
                par = lax.rem(t, 2)
                npar = 1 - par

                @pl.when(t + 1 < Tm1)
                def _():
                    pltpu.async_copy(uv_hbm.at[t + 1], uv_v.at[npar], sem_uv)

                @pl.when(t > 0)
                def _():
                    pltpu.make_async_copy(
                        dmu_v, d_out.at[pl.ds(0, E)], sem_d).wait()

                @plsc.parallel_loop(0, E, _L, unroll=4)
                def _(e):
                    iu = uv_v[par, pl.ds(e, _L)]
                    iv = uv_v[par, pl.ds(E + e, _L)]
                    dd = plsc.load_gather(x_v, [iu]) - plsc.load_gather(x_v, [iv])
                    dmu_v[pl.ds(e, _L)] = dd * mu
                pltpu.async_copy(dmu_v, d_out.at[pl.ds(t * E, E)], sem_d)

                pltpu.make_async_copy(x_v, x_out.at[0], sem_x).wait()

                @plsc.parallel_loop(0, E, _L, unroll=4)
                def _(e):
                    iu = uv_v[par, pl.ds(e, _L)]
                    iv = uv_v[par, pl.ds(E + e, _L)]
                    dm = dmu_v[pl.ds(e, _L)]
                    plsc.addupdate_scatter(x_v, [iv], dm)
                    plsc.addupdate_scatter(x_v, [iu], -dm)
                pltpu.async_copy(x_v, x_out.at[t + 1], sem_x)

                @pl.when(t + 1 < Tm1)
                def _():
                    pltpu.make_async_copy(
                        uv_hbm.at[0], uv_v.at[npar], sem_uv).wait()

                return c

            lax.fori_loop(0, Tm1, step, 0)
            pltpu.make_async_copy(dmu_v, d_out.at[pl.ds(0, E)], sem_d).wait()
            pltpu.make_async_copy(x_v, x_out.at[0], sem_x).wait()

        @pl.when(cid == 0)
        def _():
            pltpu.sync_copy(th_hbm, th_v)
            pltpu.sync_copy(mu_hbm, mu_v)
            pltpu.sync_copy(rho_hbm, rho_v)
            eps = 1.0 / (1.0 + jnp.exp(-th_v[...]))
            bias = rho_v[...] * eps
            scale = rho_v[...] / mu_v[...]
            plsc.subcore_barrier()
            base = sid * KCH
            pltpu.sync_copy(d_out.at[pl.ds(base, KCH)], x_v.at[pl.ds(0, KCH)])

            @plsc.parallel_loop(0, KCH, _L, unroll=4)
            def _(i):
                dm = x_v[pl.ds(i, _L)]
                z = bias - scale * jnp.abs(dm)
                x_v[pl.ds(i, _L)] = 1.0 / (1.0 + jnp.exp(-z))
            pltpu.sync_copy(x_v.at[pl.ds(0, KCH)], kap_out.at[pl.ds(base, KCH)])

    return body


def kernel(edges, logit_X0, theta, mu, rho):
    Tm1, E, _ = edges.shape
    N = logit_X0.shape[0]
    uv = jnp.concatenate([edges[:, :, 0], edges[:, :, 1]], axis=1)
    th16 = jnp.broadcast_to(theta.astype(jnp.float32), (_L,))
    mu16 = jnp.full((_L,), mu, jnp.float32)
    rho16 = jnp.full((_L,), rho, jnp.float32)
    X, kap, _ = _diffusion_call(Tm1, E, N)(
        uv, logit_X0.astype(jnp.float32), th16, mu16, rho16)
    return X, kap
